# baseline (speedup 1.0000x reference)
import jax
import jax.numpy as jnp
from jax.experimental import pallas as pl
from jax.experimental.pallas import tpu as pltpu

_K = 7
_PS = 10
_WS = 15
_HALF = _WS // 2
_L = _WS * _WS
_SELF = _L // 2
_EPS = 1e-8
_H = 98
_HP = _H + _PS - 1


def _n3_kernel(xe_ref, x_ref, lt_ref, out_ref, cur_ref):
    f32 = jnp.float32

    bi = jax.lax.broadcasted_iota(jnp.int32, (_H, _HP), 0)
    bu = jax.lax.broadcasted_iota(jnp.int32, (_H, _HP), 1)
    Br = ((bu >= bi) & (bu <= bi + _PS - 1)).astype(f32)
    Bc = Br.T

    def box(z, prec):
        return jnp.dot(Br, jnp.dot(z, Bc, preferred_element_type=f32,
                                   precision=prec),
                       preferred_element_type=f32, precision=prec)

    _PH = jax.lax.Precision.HIGHEST
    _PM = jax.lax.Precision.HIGHEST

    inv_temp = jnp.exp(-box(lt_ref[0], _PH) / float(_PS * _PS))

    a = xe_ref[:, 7:105, 7:105]

    def phase_a(dy, carry):
        b_rows = xe_ref[:, pl.ds(dy, _H), :]
        dl = []
        for dx in range(_WS):
            diff = a - b_rows[:, :, dx:dx + _H]
            dl.append(jnp.sum(diff * diff, axis=0))
        dchunk = jnp.stack(dl, 0)
        dp = jnp.pad(dchunk, ((0, 0), (5, 4), (5, 4)))
        ls = []
        for dx in range(_WS):
            ls.append(-box(dp[dx], _PH) * inv_temp)
        cur_ref[pl.ds(dy * _WS, _WS)] = jnp.stack(ls, 0)
        return carry

    jax.lax.fori_loop(0, _WS, phase_a, 0)
    cur_ref[_SELF] = -1e10 * inv_temp

    x_img = x_ref[:, 7:105, 7:105]
    ri = jax.lax.broadcasted_iota(jnp.int32, (_H, _H), 0)
    ci = jax.lax.broadcasted_iota(jnp.int32, (_H, _H), 1)
    rc = jnp.minimum(ri + 5, _H - 1) - jnp.maximum(ri - 4, 0) + 1
    cc = jnp.minimum(ci + 5, _H - 1) - jnp.maximum(ci - 4, 0) + 1
    inv_w = 1.0 / ((rc * cc).astype(f32) + 1e-10)

    def phase_b(k, carry):
        def mx(i, m):
            return jnp.maximum(m, jnp.max(cur_ref[pl.ds(i * _WS, _WS)], 0))

        m = jax.lax.fori_loop(0, _WS, mx, jnp.full((_H, _H), -jnp.inf, f32))

        def sm(i, s):
            return s + jnp.sum(jnp.exp(cur_ref[pl.ds(i * _WS, _WS)] - m[None]), 0)

        s = jax.lax.fori_loop(0, _WS, sm, jnp.zeros((_H, _H), f32))
        inv_s = 1.0 / s

        def agg(dy, vk):
            chunk = cur_ref[pl.ds(dy * _WS, _WS)]
            wk = jnp.exp(chunk - m[None]) * inv_s[None]
            cur_ref[pl.ds(dy * _WS, _WS)] = chunk + jnp.log(
                jnp.clip(1.0 - wk, _EPS, None))
            wkp = jnp.pad(wk, ((0, 0), (4, 5), (4, 5)))
            xsl = x_ref[:, pl.ds(dy, _H), :]
            for dx in range(_WS):
                wb = box(wkp[dx], _PM)
                vk = vk + wb[None] * xsl[:, :, dx:dx + _H]
            return vk

        vk = jax.lax.fori_loop(0, _WS, agg, jnp.zeros((3, _H, _H), f32))
        out_ref[pl.ds(3 + k * 3, 3)] = vk * inv_w[None] - x_img
        return carry

    jax.lax.fori_loop(0, _K, phase_b, 0)
    out_ref[0:3] = x_img


def kernel(x, xe, ye, log_temp):
    del ye
    xe_p = jnp.pad(xe[0], ((0, 0), (8, 8), (8, 8)))
    x_p = jnp.pad(x[0], ((0, 0), (8, 8), (8, 8)))
    lt_p = jnp.pad(log_temp[0], ((0, 0), (6, 5), (6, 5)))

    out = pl.pallas_call(
        _n3_kernel,
        out_shape=jax.ShapeDtypeStruct((3 * (_K + 1), _H, _H), jnp.float32),
        scratch_shapes=[pltpu.VMEM((_L, _H, _H), jnp.float32)],
    )(xe_p, x_p, lt_p)
    return out[None, :, 1:-1, 1:-1]


# speedup vs baseline: 2.0495x; 2.0495x over previous
import jax
import jax.numpy as jnp
from jax.experimental import pallas as pl
from jax.experimental.pallas import tpu as pltpu

_K = 7
_PS = 10
_WS = 15
_HALF = _WS // 2
_L = _WS * _WS
_SELF = _L // 2
_EPS = 1e-8
_H = 98
_HP = _H + _PS - 1


def _n3_kernel(xe_ref, x_ref, lt_ref, out_ref, cur_ref):
    f32 = jnp.float32

    bi = jax.lax.broadcasted_iota(jnp.int32, (_H, _HP), 0)
    bu = jax.lax.broadcasted_iota(jnp.int32, (_H, _HP), 1)
    Br = ((bu >= bi) & (bu <= bi + _PS - 1)).astype(f32)
    Bc = Br.T

    def box(z, prec):
        return jnp.dot(Br, jnp.dot(z, Bc, preferred_element_type=f32,
                                   precision=prec),
                       preferred_element_type=f32, precision=prec)

    _PH = jax.lax.Precision.HIGHEST
    _PM = jax.lax.Precision.DEFAULT

    inv_temp = jnp.exp(-box(lt_ref[0], _PH) / float(_PS * _PS))

    a = xe_ref[:, 7:105, 7:105]

    def phase_a(dy, carry):
        b_rows = xe_ref[:, pl.ds(dy, _H), :]
        dl = []
        for dx in range(_WS):
            diff = a - b_rows[:, :, dx:dx + _H]
            dl.append(jnp.sum(diff * diff, axis=0))
        dchunk = jnp.stack(dl, 0)
        dp = jnp.pad(dchunk, ((0, 0), (5, 4), (5, 4)))
        ls = []
        for dx in range(_WS):
            ls.append(-box(dp[dx], _PH) * inv_temp)
        cur_ref[pl.ds(dy * _WS, _WS)] = jnp.stack(ls, 0)
        return carry

    jax.lax.fori_loop(0, _WS, phase_a, 0)
    cur_ref[_SELF] = -1e10 * inv_temp

    x_img = x_ref[:, 7:105, 7:105]
    ri = jax.lax.broadcasted_iota(jnp.int32, (_H, _H), 0)
    ci = jax.lax.broadcasted_iota(jnp.int32, (_H, _H), 1)
    rc = jnp.minimum(ri + 5, _H - 1) - jnp.maximum(ri - 4, 0) + 1
    cc = jnp.minimum(ci + 5, _H - 1) - jnp.maximum(ci - 4, 0) + 1
    inv_w = 1.0 / ((rc * cc).astype(f32) + 1e-10)

    def phase_b(k, carry):
        def mx(i, m):
            return jnp.maximum(m, jnp.max(cur_ref[pl.ds(i * _WS, _WS)], 0))

        m = jax.lax.fori_loop(0, _WS, mx, jnp.full((_H, _H), -jnp.inf, f32))

        def sm(i, s):
            return s + jnp.sum(jnp.exp(cur_ref[pl.ds(i * _WS, _WS)] - m[None]), 0)

        s = jax.lax.fori_loop(0, _WS, sm, jnp.zeros((_H, _H), f32))
        inv_s = 1.0 / s

        def agg(dy, vk):
            chunk = cur_ref[pl.ds(dy * _WS, _WS)]
            wk = jnp.exp(chunk - m[None]) * inv_s[None]
            cur_ref[pl.ds(dy * _WS, _WS)] = chunk + jnp.log(
                jnp.clip(1.0 - wk, _EPS, None))
            wkp = jnp.pad(wk, ((0, 0), (4, 5), (4, 5)))
            xsl = x_ref[:, pl.ds(dy, _H), :]
            for dx in range(_WS):
                wb = box(wkp[dx], _PM)
                vk = vk + wb[None] * xsl[:, :, dx:dx + _H]
            return vk

        vk = jax.lax.fori_loop(0, _WS, agg, jnp.zeros((3, _H, _H), f32))
        out_ref[pl.ds(3 + k * 3, 3)] = vk * inv_w[None] - x_img
        return carry

    jax.lax.fori_loop(0, _K, phase_b, 0)
    out_ref[0:3] = x_img


def kernel(x, xe, ye, log_temp):
    del ye
    xe_p = jnp.pad(xe[0], ((0, 0), (8, 8), (8, 8)))
    x_p = jnp.pad(x[0], ((0, 0), (8, 8), (8, 8)))
    lt_p = jnp.pad(log_temp[0], ((0, 0), (6, 5), (6, 5)))

    out = pl.pallas_call(
        _n3_kernel,
        out_shape=jax.ShapeDtypeStruct((3 * (_K + 1), _H, _H), jnp.float32),
        scratch_shapes=[pltpu.VMEM((_L, _H, _H), jnp.float32)],
    )(xe_p, x_p, lt_p)
    return out[None, :, 1:-1, 1:-1]


# speedup vs baseline: 2.2205x; 1.0834x over previous
import jax
import jax.numpy as jnp
from jax.experimental import pallas as pl
from jax.experimental.pallas import tpu as pltpu

_K = 7
_PS = 10
_WS = 15
_HALF = _WS // 2
_L = _WS * _WS
_SELF = _L // 2
_EPS = 1e-8
_H = 98
_HP = _H + _PS - 1


def _n3_kernel(xe_ref, x_ref, lt_ref, out_ref, cur_ref, x15_ref):
    f32 = jnp.float32

    bi = jax.lax.broadcasted_iota(jnp.int32, (_H, _HP), 0)
    bu = jax.lax.broadcasted_iota(jnp.int32, (_H, _HP), 1)
    Br = ((bu >= bi) & (bu <= bi + _PS - 1)).astype(f32)
    Bc = Br.T

    def box_mm(z):
        return jnp.dot(Br, jnp.dot(z, Bc, preferred_element_type=f32),
                       preferred_element_type=f32)

    ltp = lt_ref[0]
    ltc = ltp[:, 0:_H]
    for t in range(1, _PS):
        ltc = ltc + ltp[:, t:t + _H]
    ltr = ltc[0:_H]
    for s in range(1, _PS):
        ltr = ltr + ltc[s:s + _H]
    inv_temp = jnp.exp(-ltr / float(_PS * _PS))

    for dy in range(_WS):
        x15_ref[dy] = x_ref[:, dy:dy + _H, :]

    a = xe_ref[:, 7:105, 7:105]

    def phase_a(dy, carry):
        b_rows = xe_ref[:, pl.ds(dy, _H), :]
        dl = []
        for dx in range(_WS):
            diff = a - b_rows[:, :, dx:dx + _H]
            dl.append(jnp.sum(diff * diff, axis=0))
        dchunk = jnp.stack(dl, 0)
        dp = jnp.pad(dchunk, ((0, 0), (5, 4), (5, 4)))
        dc = dp[:, :, 0:_H]
        for t in range(1, _PS):
            dc = dc + dp[:, :, t:t + _H]
        Dr = dc[:, 0:_H]
        for s in range(1, _PS):
            Dr = Dr + dc[:, s:s + _H]
        cur_ref[pl.ds(dy * _WS, _WS)] = -Dr * inv_temp[None]
        return carry

    jax.lax.fori_loop(0, _WS, phase_a, 0)
    cur_ref[_SELF] = -1e10 * inv_temp

    x_img = x_ref[:, 7:105, 7:105]
    ri = jax.lax.broadcasted_iota(jnp.int32, (_H, _H), 0)
    ci = jax.lax.broadcasted_iota(jnp.int32, (_H, _H), 1)
    rc = jnp.minimum(ri + 5, _H - 1) - jnp.maximum(ri - 4, 0) + 1
    cc = jnp.minimum(ci + 5, _H - 1) - jnp.maximum(ci - 4, 0) + 1
    inv_w = 1.0 / ((rc * cc).astype(f32) + 1e-10)

    def phase_b(k, carry):
        def mx(i, m):
            return jnp.maximum(m, jnp.max(cur_ref[pl.ds(i * _WS, _WS)], 0))

        m = jax.lax.fori_loop(0, _WS, mx, jnp.full((_H, _H), -jnp.inf, f32))

        def sm(i, s):
            return s + jnp.sum(jnp.exp(cur_ref[pl.ds(i * _WS, _WS)] - m[None]), 0)

        s = jax.lax.fori_loop(0, _WS, sm, jnp.zeros((_H, _H), f32))
        inv_s = 1.0 / s

        def agg(dy, vk):
            chunk = cur_ref[pl.ds(dy * _WS, _WS)]
            wk = jnp.exp(chunk - m[None]) * inv_s[None]
            cur_ref[pl.ds(dy * _WS, _WS)] = chunk + jnp.log(
                jnp.clip(1.0 - wk, _EPS, None))
            wkp = jnp.pad(wk, ((0, 0), (4, 5), (4, 5)))
            xsl = x15_ref[dy]
            for dx in range(_WS):
                wb = box_mm(wkp[dx])
                vk = vk + wb[None] * xsl[:, :, dx:dx + _H]
            return vk

        vk = jax.lax.fori_loop(0, _WS, agg, jnp.zeros((3, _H, _H), f32))
        out_ref[pl.ds(3 + k * 3, 3)] = vk * inv_w[None] - x_img
        return carry

    jax.lax.fori_loop(0, _K, phase_b, 0)
    out_ref[0:3] = x_img


def kernel(x, xe, ye, log_temp):
    del ye
    xe_p = jnp.pad(xe[0], ((0, 0), (8, 8), (8, 8)))
    x_p = jnp.pad(x[0], ((0, 0), (8, 8), (8, 8)))
    lt_p = jnp.pad(log_temp[0], ((0, 0), (6, 5), (6, 5)))

    out = pl.pallas_call(
        _n3_kernel,
        out_shape=jax.ShapeDtypeStruct((3 * (_K + 1), _H, _H), jnp.float32),
        scratch_shapes=[
            pltpu.VMEM((_L, _H, _H), jnp.float32),
            pltpu.VMEM((_WS, 3, _H, 112), jnp.float32),
        ],
    )(xe_p, x_p, lt_p)
    return out[None, :, 1:-1, 1:-1]


# speedup vs baseline: 3.9381x; 1.7735x over previous
import jax
import jax.numpy as jnp
from jax.experimental import pallas as pl
from jax.experimental.pallas import tpu as pltpu

_K = 7
_PS = 10
_WS = 15
_HALF = _WS // 2
_L = _WS * _WS
_SELF = _L // 2
_EPS = 1e-8
_H = 98
_HP = _H + _PS - 1


def _n3_kernel(xe_ref, x_ref, lt_ref, out_ref, cur_ref, x15_ref):
    f32 = jnp.float32

    bi = jax.lax.broadcasted_iota(jnp.int32, (_H, _HP), 0)
    bu = jax.lax.broadcasted_iota(jnp.int32, (_H, _HP), 1)
    Br = ((bu >= bi) & (bu <= bi + _PS - 1)).astype(f32)
    Bc = Br.T

    bi2 = jax.lax.broadcasted_iota(jnp.int32, (_H, 112), 0)
    bu2 = jax.lax.broadcasted_iota(jnp.int32, (_H, 112), 1)
    Br2 = ((bu2 >= bi2) & (bu2 <= bi2 + _PS - 1)).astype(f32)
    Bc2 = Br2.T

    ltp = lt_ref[0]
    ltc = ltp[:, 0:_H]
    for t in range(1, _PS):
        ltc = ltc + ltp[:, t:t + _H]
    ltr = ltc[0:_H]
    for s in range(1, _PS):
        ltr = ltr + ltc[s:s + _H]
    inv_temp = jnp.exp(-ltr / float(_PS * _PS))

    for dy in range(_WS):
        x15_ref[dy] = x_ref[:, dy:dy + _H, :]

    a = xe_ref[:, 7:105, 7:105]

    def phase_a(dy, carry):
        b_rows = xe_ref[:, pl.ds(dy, _H), :]
        dl = []
        for dx in range(_WS):
            diff = a - b_rows[:, :, dx:dx + _H]
            dl.append(jnp.sum(diff * diff, axis=0))
        dchunk = jnp.stack(dl, 0)
        dp = jnp.pad(dchunk, ((0, 0), (5, 4), (5, 4)))
        dc = dp[:, :, 0:_H]
        for t in range(1, _PS):
            dc = dc + dp[:, :, t:t + _H]
        Dr = dc[:, 0:_H]
        for s in range(1, _PS):
            Dr = Dr + dc[:, s:s + _H]
        cur_ref[pl.ds(dy * _WS, _WS)] = -Dr * inv_temp[None]
        return carry

    jax.lax.fori_loop(0, _WS, phase_a, 0)
    cur_ref[_SELF] = -1e10 * inv_temp

    x_img = x_ref[:, 7:105, 7:105]
    ri = jax.lax.broadcasted_iota(jnp.int32, (_H, _H), 0)
    ci = jax.lax.broadcasted_iota(jnp.int32, (_H, _H), 1)
    rc = jnp.minimum(ri + 5, _H - 1) - jnp.maximum(ri - 4, 0) + 1
    cc = jnp.minimum(ci + 5, _H - 1) - jnp.maximum(ci - 4, 0) + 1
    inv_w = 1.0 / ((rc * cc).astype(f32) + 1e-10)

    def phase_b(k, carry):
        def mx(i, m):
            return jnp.maximum(m, jnp.max(cur_ref[pl.ds(i * _WS, _WS)], 0))

        m = jax.lax.fori_loop(0, _WS, mx, jnp.full((_H, _H), -jnp.inf, f32))

        def sm(i, s):
            return s + jnp.sum(jnp.exp(cur_ref[pl.ds(i * _WS, _WS)] - m[None]), 0)

        s = jax.lax.fori_loop(0, _WS, sm, jnp.zeros((_H, _H), f32))
        inv_s = 1.0 / s

        def agg(dy, vk):
            chunk = cur_ref[pl.ds(dy * _WS, _WS)]
            wk = jnp.exp(chunk - m[None]) * inv_s[None]
            cur_ref[pl.ds(dy * _WS, _WS)] = chunk + jnp.log(
                jnp.clip(1.0 - wk, _EPS, None))
            wkp = jnp.pad(wk, ((0, 0), (4, 10), (4, 10)))
            cb = jnp.dot(wkp.reshape(_WS * 112, 112), Bc2,
                         preferred_element_type=f32).reshape(_WS, 112, _H)
            xsl = x15_ref[dy]
            for dx in range(_WS):
                wb = jnp.dot(Br2, cb[dx], preferred_element_type=f32)
                vk = vk + wb[None] * xsl[:, :, dx:dx + _H]
            return vk

        vk = jax.lax.fori_loop(0, _WS, agg, jnp.zeros((3, _H, _H), f32))
        out_ref[pl.ds(3 + k * 3, 3)] = vk * inv_w[None] - x_img
        return carry

    jax.lax.fori_loop(0, _K, phase_b, 0)
    out_ref[0:3] = x_img


def kernel(x, xe, ye, log_temp):
    del ye
    xe_p = jnp.pad(xe[0], ((0, 0), (8, 8), (8, 8)))
    x_p = jnp.pad(x[0], ((0, 0), (8, 8), (8, 8)))
    lt_p = jnp.pad(log_temp[0], ((0, 0), (6, 5), (6, 5)))

    out = pl.pallas_call(
        _n3_kernel,
        out_shape=jax.ShapeDtypeStruct((3 * (_K + 1), _H, _H), jnp.float32),
        scratch_shapes=[
            pltpu.VMEM((_L, _H, _H), jnp.float32),
            pltpu.VMEM((_WS, 3, _H, 112), jnp.float32),
        ],
    )(xe_p, x_p, lt_p)
    return out[None, :, 1:-1, 1:-1]
